# Initial kernel scaffold; baseline (speedup 1.0000x reference)
import functools

import jax
import jax.numpy as jnp
from jax import lax
from jax.experimental import pallas as pl
from jax.experimental.pallas import tpu as pltpu
from jax.experimental.pallas import tpu_sc as plsc

_N = 10000
_E = 320000
_D = 128

_NC = 2
_NS = 16
_NW = _NC * _NS
_EPW = _E // _NW
_CH = 80
_NCHUNK = _EPW // _CH
_RPT = _N // _NS
_ZR = 125


def _mm_body(x_ref, w_ref, o_ref):
    o_ref[...] = jnp.dot(x_ref[...], w_ref[...],
                         preferred_element_type=jnp.float32)


def _matmul(x, W):
    return pl.pallas_call(
        _mm_body,
        grid=(10,),
        in_specs=[
            pl.BlockSpec((1000, _D), lambda i: (i, 0)),
            pl.BlockSpec((_D, _D), lambda i: (0, 0)),
        ],
        out_specs=pl.BlockSpec((1000, _D), lambda i: (i, 0)),
        out_shape=jax.ShapeDtypeStruct((_N, _D), jnp.float32),
    )(x, W)


def _add_body(a_ref, b_ref, o_ref):
    o_ref[...] = a_ref[...] + b_ref[...]


def _combine(p0, p1):
    return pl.pallas_call(
        _add_body,
        grid=(10,),
        in_specs=[
            pl.BlockSpec((1000, _D), lambda i: (i, 0)),
            pl.BlockSpec((1000, _D), lambda i: (i, 0)),
        ],
        out_specs=pl.BlockSpec((1000, _D), lambda i: (i, 0)),
        out_shape=jax.ShapeDtypeStruct((_N, _D), jnp.float32),
    )(p0, p1)


_mesh = plsc.VectorSubcoreMesh(core_axis_name="c", subcore_axis_name="s")


@functools.partial(
    pl.kernel,
    mesh=_mesh,
    out_type=jax.ShapeDtypeStruct((_NC, _N, _D), jnp.float32),
    scratch_types=[
        pltpu.VMEM((_CH,), jnp.int32),
        pltpu.VMEM((_CH,), jnp.int32),
        pltpu.VMEM((_CH,), jnp.float32),
        pltpu.VMEM((_CH, _D), jnp.float32),
        pltpu.VMEM((_ZR, _D), jnp.float32),
        pltpu.VMEM_SHARED((_N, _D), jnp.float32),
        pltpu.SemaphoreType.DMA,
    ],
)
def _sc_spmm(sup_hbm, src_hbm, dst_hbm, ev_hbm, out_hbm,
             srcv, dstv, evv, rows, zbuf, acc, sem):
    c = lax.axis_index("c")
    s = lax.axis_index("s")
    wid = c * _NS + s

    def zb_body(i, carry):
        for b in range(_D // 16):
            zbuf[i, pl.ds(b * 16, 16)] = jnp.zeros((16,), jnp.float32)
        return carry

    lax.fori_loop(0, _ZR, zb_body, 0)
    for kz in range(_RPT // _ZR):
        pltpu.sync_copy(zbuf, acc.at[pl.ds(s * _RPT + kz * _ZR, _ZR)])
    plsc.subcore_barrier()

    def chunk_body(k, carry):
        base = wid * _EPW + k * _CH
        pltpu.sync_copy(src_hbm.at[pl.ds(base, _CH)], srcv)
        pltpu.sync_copy(dst_hbm.at[pl.ds(base, _CH)], dstv)
        pltpu.sync_copy(ev_hbm.at[pl.ds(base, _CH)], evv)
        pltpu.async_copy(sup_hbm.at[srcv], rows, sem).wait()

        def mul_body(e, inner):
            v = evv[e]
            for b in range(_D // 16):
                sl = rows[e, pl.ds(b * 16, 16)]
                rows[e, pl.ds(b * 16, 16)] = sl * v
            return inner

        lax.fori_loop(0, _CH, mul_body, 0)
        pltpu.sync_copy(rows, acc.at[dstv], add=True)
        return carry

    lax.fori_loop(0, _NCHUNK, chunk_body, 0)
    plsc.subcore_barrier()

    pltpu.sync_copy(acc.at[pl.ds(s * _RPT, _RPT)],
                    out_hbm.at[c, pl.ds(s * _RPT, _RPT)])


def kernel(x, edge_index, edge_vals, W):
    support = _matmul(x, W)
    dst = edge_index[0]
    src = edge_index[1]
    partials = _sc_spmm(support, src, dst, edge_vals)
    return _combine(partials[0], partials[1])


# speedup vs baseline: 4.3765x; 4.3765x over previous
import functools

import jax
import jax.numpy as jnp
from jax import lax
from jax.experimental import pallas as pl
from jax.experimental.pallas import tpu as pltpu
from jax.experimental.pallas import tpu_sc as plsc

_N = 10000
_E = 320000
_D = 128

_NC = 2
_NS = 16
_NW = _NC * _NS
_EPW = _E // _NW
_CH = 80
_NCHUNK = _EPW // _CH
_SLAB = 624
_ZB = 208


def _mm_body(x_ref, w_ref, o_ref):
    o_ref[...] = jnp.dot(x_ref[...], w_ref[...],
                         preferred_element_type=jnp.float32)


def _matmul(x, W):
    return pl.pallas_call(
        _mm_body,
        grid=(10,),
        in_specs=[
            pl.BlockSpec((1000, _D), lambda i: (i, 0)),
            pl.BlockSpec((_D, _D), lambda i: (0, 0)),
        ],
        out_specs=pl.BlockSpec((1000, _D), lambda i: (i, 0)),
        out_shape=jax.ShapeDtypeStruct((_N, _D), jnp.float32),
    )(x, W)


def _add_body(a_ref, b_ref, o_ref):
    o_ref[...] = a_ref[...] + b_ref[...]


def _combine(p0, p1):
    return pl.pallas_call(
        _add_body,
        grid=(10,),
        in_specs=[
            pl.BlockSpec((1000, _D), lambda i: (i, 0)),
            pl.BlockSpec((1000, _D), lambda i: (i, 0)),
        ],
        out_specs=pl.BlockSpec((1000, _D), lambda i: (i, 0)),
        out_shape=jax.ShapeDtypeStruct((_N, _D), jnp.float32),
    )(p0, p1)


_mesh = plsc.VectorSubcoreMesh(core_axis_name="c", subcore_axis_name="s")


@functools.partial(
    pl.kernel,
    mesh=_mesh,
    out_type=jax.ShapeDtypeStruct((_NC, _N, _D), jnp.float32),
    scratch_types=[
        pltpu.VMEM((_CH,), jnp.int32),
        pltpu.VMEM((_CH,), jnp.int32),
        pltpu.VMEM((_CH,), jnp.float32),
        pltpu.VMEM((_CH, _D), jnp.float32),
        pltpu.VMEM((_ZB, _D), jnp.float32),
        pltpu.VMEM_SHARED((_N, _D), jnp.float32),
        pltpu.SemaphoreType.DMA,
    ],
)
def _sc_spmm(sup_hbm, src_hbm, dst_hbm, ev_hbm, out_hbm,
             srcv, dstv, evv, rows, zbuf, acc, sem):
    c = lax.axis_index("c")
    s = lax.axis_index("s")
    wid = c * _NS + s

    def zb_body(i, carry):
        for b in range(_D // 16):
            zbuf[i, pl.ds(b * 16, 16)] = jnp.zeros((16,), jnp.float32)
        return carry

    lax.fori_loop(0, _ZB, zb_body, 0)
    for kz in range(_SLAB // _ZB):
        pltpu.sync_copy(zbuf, acc.at[pl.ds(s * _SLAB + kz * _ZB, _ZB)])

    @pl.when(s == _NS - 1)
    def _zero_tail():
        pltpu.sync_copy(zbuf.at[pl.ds(0, _N - _NS * _SLAB)],
                        acc.at[pl.ds(_NS * _SLAB, _N - _NS * _SLAB)])

    plsc.subcore_barrier()

    def chunk_body(k, carry):
        base = wid * _EPW + k * _CH
        pltpu.sync_copy(src_hbm.at[pl.ds(base, _CH)], srcv)
        pltpu.sync_copy(dst_hbm.at[pl.ds(base, _CH)], dstv)
        pltpu.sync_copy(ev_hbm.at[pl.ds(base, _CH)], evv)
        pltpu.async_copy(sup_hbm.at[srcv], rows, sem).wait()

        def mul_body(g, inner):
            evg = evv[pl.ds(g * 16, 16)]
            for j in range(16):
                v = evg[j]
                e = g * 16 + j
                for b in range(_D // 16):
                    sl = rows[e, pl.ds(b * 16, 16)]
                    rows[e, pl.ds(b * 16, 16)] = sl * v
            return inner

        lax.fori_loop(0, _CH // 16, mul_body, 0)
        pltpu.sync_copy(rows, acc.at[dstv], add=True)
        return carry

    lax.fori_loop(0, _NCHUNK, chunk_body, 0)
    plsc.subcore_barrier()

    pltpu.sync_copy(acc.at[pl.ds(s * _SLAB, _SLAB)],
                    out_hbm.at[c, pl.ds(s * _SLAB, _SLAB)])

    @pl.when(s == _NS - 1)
    def _copy_tail():
        pltpu.sync_copy(acc.at[pl.ds(_NS * _SLAB, _N - _NS * _SLAB)],
                        out_hbm.at[c, pl.ds(_NS * _SLAB, _N - _NS * _SLAB)])


def kernel(x, edge_index, edge_vals, W):
    support = _matmul(x, W)
    dst = edge_index[0]
    src = edge_index[1]
    partials = _sc_spmm(support, src, dst, edge_vals)
    return _combine(partials[0], partials[1])
